# baseline (speedup 1.0000x reference)
import functools

import jax
import jax.numpy as jnp
from jax import lax
from jax.experimental import pallas as pl
from jax.experimental.pallas import tpu as pltpu
from jax.experimental.pallas import tpu_sc as plsc

_N = 10000
_E = 320000
_D = 128
_H = 128
_C = 16
_G = 64
_EPS = 1e-5

_NC = 2
_NS = 16
_NW = _NC * _NS
_EPW = _E // _NW
_CH = 128
_NCHUNK = 80
_EPWP = _NCHUNK * _CH
_GRP = 8
_NP = 10240
_RPT = _NP // _NS

_mesh = plsc.VectorSubcoreMesh(core_axis_name="c", subcore_axis_name="s")


@functools.partial(
    pl.kernel,
    out_type=jax.ShapeDtypeStruct((_NC, _NP, _H), jnp.float32),
    mesh=_mesh,
    scratch_types=[
        pltpu.VMEM((_NCHUNK, _CH), jnp.int32),
        pltpu.VMEM((_CH, _H), jnp.float32),
        pltpu.VMEM_SHARED((_NP, _H), jnp.float32),
    ],
)
def _sc_degree(dst_h, ones_h, zer_h, out_h, idx_v, ones_v, deg_sh):
    c = lax.axis_index("c")
    s = lax.axis_index("s")
    w = c * _NS + s
    pltpu.sync_copy(zer_h, deg_sh.at[pl.ds(s * _RPT, _RPT)])
    pltpu.sync_copy(ones_h, ones_v)
    pltpu.sync_copy(dst_h.at[w], idx_v)
    plsc.subcore_barrier()

    def step(j, carry):
        pltpu.sync_copy(ones_v, deg_sh.at[idx_v.at[j]], add=True)
        return carry

    lax.fori_loop(0, _NCHUNK, step, 0)
    plsc.subcore_barrier()
    pltpu.sync_copy(deg_sh.at[pl.ds(s * _RPT, _RPT)],
                    out_h.at[c, pl.ds(s * _RPT, _RPT)])


@functools.partial(
    pl.kernel,
    out_type=jax.ShapeDtypeStruct((_NC, _NP, _H), jnp.float32),
    mesh=_mesh,
    scratch_types=[
        pltpu.VMEM((_NCHUNK, _CH), jnp.int32),
        pltpu.VMEM((_GRP, _CH), jnp.int32),
        pltpu.VMEM((_CH, _H), jnp.float32),
        pltpu.VMEM((_CH, _H), jnp.float32),
        pltpu.VMEM_SHARED((_NP, _H), jnp.float32),
        pltpu.SemaphoreType.DMA,
        pltpu.SemaphoreType.DMA,
    ],
)
def _sc_segsum(xs_h, src_h, dst_h, zer_h, out_h,
               sidx, dring, rows0, rows1, acc_sh, gsem0, gsem1):
    c = lax.axis_index("c")
    s = lax.axis_index("s")
    w = c * _NS + s
    pltpu.sync_copy(zer_h, acc_sh.at[pl.ds(s * _RPT, _RPT)])
    pltpu.sync_copy(src_h.at[w], sidx)
    plsc.subcore_barrier()

    bufs = (rows0, rows1)
    sems = (gsem0, gsem1)

    def step(k, carry):
        base = k * _GRP
        pltpu.sync_copy(dst_h.at[w, pl.ds(base, _GRP)], dring)
        descs = [None] * _GRP
        descs[0] = pltpu.async_copy(xs_h.at[sidx.at[base]], bufs[0], sems[0])
        descs[1] = pltpu.async_copy(xs_h.at[sidx.at[base + 1]], bufs[1],
                                    sems[1])
        for t in range(_GRP):
            b = t % 2
            descs[t].wait()
            pltpu.sync_copy(bufs[b], acc_sh.at[dring.at[t]], add=True)
            if t + 2 < _GRP:
                descs[t + 2] = pltpu.async_copy(
                    xs_h.at[sidx.at[base + t + 2]], bufs[b], sems[b])
        return carry

    lax.fori_loop(0, _NCHUNK // _GRP, step, 0)
    plsc.subcore_barrier()
    pltpu.sync_copy(acc_sh.at[pl.ds(s * _RPT, _RPT)],
                    out_h.at[c, pl.ds(s * _RPT, _RPT)])


def _dinv_from_partials(degp):
    deg = degp[0, :_N, 0:1] + degp[1, :_N, 0:1] + 1.0
    return lax.rsqrt(deg)


def _tc_pre_body(x_ref, w1_ref, g1_ref, degp_ref, u1_ref, xs1_ref):
    s1 = g1_ref[:] * (1.0 / jnp.sqrt(1.0 + _EPS))
    w1f = w1_ref[:] * s1[None, :]
    u1 = jnp.dot(x_ref[:], w1f, preferred_element_type=jnp.float32)
    dinv = _dinv_from_partials(degp_ref[:])
    u1_ref[:] = u1
    xs1_ref[:] = u1 * dinv


def _tc_mid_body(a_ref, u1_ref, degp_ref, w2_ref, b1_ref, g1_ref, bb1_ref,
                 u2_ref, xs2_ref):
    dinv = _dinv_from_partials(degp_ref[:])
    s1 = g1_ref[:] * (1.0 / jnp.sqrt(1.0 + _EPS))
    b1f = b1_ref[:] * s1 + bb1_ref[:]
    agg = dinv * (a_ref[0, :_N] + a_ref[1, :_N]) + (dinv * dinv) * u1_ref[:]
    h = jnp.maximum(agg + b1f[None, :], 0.0)
    u2 = jnp.dot(h, w2_ref[:], preferred_element_type=jnp.float32)
    u2_ref[:] = u2
    xs2_ref[:] = u2 * dinv


def _tc_post_body(b_ref, u2_ref, degp_ref, b2_ref, bat_ref,
                  cw1_ref, cb1_ref, cg_ref, cbb_ref, cw2_ref, cb2_ref,
                  pred_ref, rep_ref):
    dinv = _dinv_from_partials(degp_ref[:])
    emb = (dinv * (b_ref[0, :_N] + b_ref[1, :_N]) + (dinv * dinv) * u2_ref[:]
           + b2_ref[:][None, :])
    seg = lax.broadcasted_iota(jnp.int32, (_G, _N), 0)
    oh = (seg == bat_ref[:]).astype(jnp.float32)
    counts = jnp.sum(oh, axis=1, keepdims=True)
    sums = jnp.dot(oh, emb, preferred_element_type=jnp.float32)
    rep = sums / jnp.maximum(counts, 1.0)
    rep = rep / jnp.sqrt(jnp.sum(rep * rep, axis=1, keepdims=True))
    cs = cg_ref[:] * (1.0 / jnp.sqrt(1.0 + _EPS))
    z = jnp.dot(rep, cw1_ref[:], preferred_element_type=jnp.float32)
    z = jnp.maximum((z + cb1_ref[:][None, :]) * cs[None, :] + cbb_ref[:][None, :], 0.0)
    z = jnp.dot(z, cw2_ref[:], preferred_element_type=jnp.float32) + cb2_ref[:][None, :]
    m = jnp.max(z, axis=1, keepdims=True)
    lse = m + jnp.log(jnp.sum(jnp.exp(z - m), axis=1, keepdims=True))
    pred_ref[:] = z - lse
    rep_ref[:] = rep


_tc_pre = pl.pallas_call(
    _tc_pre_body,
    out_shape=[jax.ShapeDtypeStruct((_N, _H), jnp.float32),
               jax.ShapeDtypeStruct((_N, _H), jnp.float32)],
)
_tc_mid = pl.pallas_call(
    _tc_mid_body,
    out_shape=[jax.ShapeDtypeStruct((_N, _H), jnp.float32),
               jax.ShapeDtypeStruct((_N, _H), jnp.float32)],
)
_tc_post = pl.pallas_call(
    _tc_post_body,
    out_shape=[jax.ShapeDtypeStruct((_G, _C), jnp.float32),
               jax.ShapeDtypeStruct((_G, _H), jnp.float32)],
)


def kernel(x, edge_index, batch, W1, b1, bn1_g, bn1_b, W2, b2,
           cW1, cb1, cbn_g, cbn_b, cW2, cb2):
    pad = _EPWP - _EPW
    srcw = edge_index[0].reshape(_NW, _EPW)
    dstw = edge_index[1].reshape(_NW, _EPW)
    src3 = jnp.pad(srcw, ((0, 0), (0, pad))).reshape(_NW, _NCHUNK, _CH)
    dst3 = jnp.pad(dstw, ((0, 0), (0, pad)),
                   constant_values=_N).reshape(_NW, _NCHUNK, _CH)
    bat2 = batch.reshape(1, _N)
    ones_row = jnp.ones((_CH, _H), jnp.float32)
    zer_row = jnp.zeros((_RPT, _H), jnp.float32)

    degp = _sc_degree(dst3, ones_row, zer_row)
    u1, xs1 = _tc_pre(x, W1, bn1_g, degp)
    a_part = _sc_segsum(xs1, src3, dst3, zer_row)
    u2, xs2 = _tc_mid(a_part, u1, degp, W2, b1, bn1_g, bn1_b)
    b_part = _sc_segsum(xs2, src3, dst3, zer_row)
    pred, rep = _tc_post(b_part, u2, degp, b2, bat2,
                         cW1, cb1, cbn_g, cbn_b, cW2, cb2)
    return (pred, rep)


# speedup vs baseline: 1.9388x; 1.9388x over previous
import functools

import jax
import jax.numpy as jnp
from jax import lax
from jax.experimental import pallas as pl
from jax.experimental.pallas import tpu as pltpu
from jax.experimental.pallas import tpu_sc as plsc

_N = 10000
_E = 320000
_D = 128
_H = 128
_C = 16
_G = 64
_EPS = 1e-5

_NC = 2
_NS = 16
_NW = _NC * _NS
_EPW = _E // _NW
_CH = 125
_NCHUNK = 80
_NP = 10240
_RPT = _NP // _NS

_mesh = plsc.VectorSubcoreMesh(core_axis_name="c", subcore_axis_name="s")


@functools.partial(
    pl.kernel,
    out_type=jax.ShapeDtypeStruct((_NC, _NP, _H), jnp.float32),
    mesh=_mesh,
    scratch_types=[
        pltpu.VMEM((_NCHUNK, _CH), jnp.int32),
        pltpu.VMEM((_CH, _H), jnp.float32),
        pltpu.VMEM_SHARED((_NP, _H), jnp.float32),
    ],
)
def _sc_degree(dst_h, ones_h, zer_h, out_h, idx_v, ones_v, deg_sh):
    c = lax.axis_index("c")
    s = lax.axis_index("s")
    w = c * _NS + s
    pltpu.sync_copy(zer_h, deg_sh.at[pl.ds(s * _RPT, _RPT)])
    pltpu.sync_copy(ones_h, ones_v)
    pltpu.sync_copy(dst_h.at[w], idx_v)
    plsc.subcore_barrier()

    def step(j, carry):
        pltpu.sync_copy(ones_v, deg_sh.at[idx_v.at[j]], add=True)
        return carry

    lax.fori_loop(0, _NCHUNK, step, 0)
    plsc.subcore_barrier()
    pltpu.sync_copy(deg_sh.at[pl.ds(s * _RPT, _RPT)],
                    out_h.at[c, pl.ds(s * _RPT, _RPT)])


@functools.partial(
    pl.kernel,
    out_type=jax.ShapeDtypeStruct((_NC, _NP, _H), jnp.float32),
    mesh=_mesh,
    scratch_types=[
        pltpu.VMEM((_NCHUNK, _CH), jnp.int32),
        pltpu.VMEM((_NCHUNK, _CH), jnp.int32),
        pltpu.VMEM((_CH, _H), jnp.float32),
        pltpu.VMEM_SHARED((_NP, _H), jnp.float32),
        pltpu.SemaphoreType.DMA,
    ],
)
def _sc_segsum(xs_h, src_h, dst_h, zer_h, out_h,
               sidx, didx, rows, acc_sh, sem):
    c = lax.axis_index("c")
    s = lax.axis_index("s")
    w = c * _NS + s
    pltpu.sync_copy(zer_h, acc_sh.at[pl.ds(s * _RPT, _RPT)])
    pltpu.sync_copy(src_h.at[w], sidx)
    pltpu.sync_copy(dst_h.at[w], didx)
    plsc.subcore_barrier()

    def step(j, carry):
        pltpu.async_copy(xs_h.at[sidx.at[j]], rows, sem).wait()
        pltpu.sync_copy(rows, acc_sh.at[didx.at[j]], add=True)
        return carry

    lax.fori_loop(0, _NCHUNK, step, 0)
    plsc.subcore_barrier()
    pltpu.sync_copy(acc_sh.at[pl.ds(s * _RPT, _RPT)],
                    out_h.at[c, pl.ds(s * _RPT, _RPT)])


def _dinv_from_partials(degp):
    deg = degp[0, :_N, 0:1] + degp[1, :_N, 0:1] + 1.0
    return lax.rsqrt(deg)


def _tc_pre_body(x_ref, w1_ref, g1_ref, degp_ref, u1_ref, xs1_ref):
    s1 = g1_ref[:] * (1.0 / jnp.sqrt(1.0 + _EPS))
    w1f = w1_ref[:] * s1[None, :]
    u1 = jnp.dot(x_ref[:], w1f, preferred_element_type=jnp.float32)
    dinv = _dinv_from_partials(degp_ref[:])
    u1_ref[:] = u1
    xs1_ref[:] = u1 * dinv


def _tc_mid_body(a_ref, u1_ref, degp_ref, w2_ref, b1_ref, g1_ref, bb1_ref,
                 u2_ref, xs2_ref):
    dinv = _dinv_from_partials(degp_ref[:])
    s1 = g1_ref[:] * (1.0 / jnp.sqrt(1.0 + _EPS))
    b1f = b1_ref[:] * s1 + bb1_ref[:]
    agg = dinv * (a_ref[0, :_N] + a_ref[1, :_N]) + (dinv * dinv) * u1_ref[:]
    h = jnp.maximum(agg + b1f[None, :], 0.0)
    u2 = jnp.dot(h, w2_ref[:], preferred_element_type=jnp.float32)
    u2_ref[:] = u2
    xs2_ref[:] = u2 * dinv


def _tc_post_body(b_ref, u2_ref, degp_ref, b2_ref, bat_ref,
                  cw1_ref, cb1_ref, cg_ref, cbb_ref, cw2_ref, cb2_ref,
                  pred_ref, rep_ref):
    dinv = _dinv_from_partials(degp_ref[:])
    emb = (dinv * (b_ref[0, :_N] + b_ref[1, :_N]) + (dinv * dinv) * u2_ref[:]
           + b2_ref[:][None, :])
    seg = lax.broadcasted_iota(jnp.int32, (_G, _N), 0)
    oh = (seg == bat_ref[:]).astype(jnp.float32)
    counts = jnp.sum(oh, axis=1, keepdims=True)
    sums = jnp.dot(oh, emb, preferred_element_type=jnp.float32)
    rep = sums / jnp.maximum(counts, 1.0)
    rep = rep / jnp.sqrt(jnp.sum(rep * rep, axis=1, keepdims=True))
    cs = cg_ref[:] * (1.0 / jnp.sqrt(1.0 + _EPS))
    z = jnp.dot(rep, cw1_ref[:], preferred_element_type=jnp.float32)
    z = jnp.maximum((z + cb1_ref[:][None, :]) * cs[None, :] + cbb_ref[:][None, :], 0.0)
    z = jnp.dot(z, cw2_ref[:], preferred_element_type=jnp.float32) + cb2_ref[:][None, :]
    m = jnp.max(z, axis=1, keepdims=True)
    lse = m + jnp.log(jnp.sum(jnp.exp(z - m), axis=1, keepdims=True))
    pred_ref[:] = z - lse
    rep_ref[:] = rep


_tc_pre = pl.pallas_call(
    _tc_pre_body,
    out_shape=[jax.ShapeDtypeStruct((_N, _H), jnp.float32),
               jax.ShapeDtypeStruct((_N, _H), jnp.float32)],
)
_tc_mid = pl.pallas_call(
    _tc_mid_body,
    out_shape=[jax.ShapeDtypeStruct((_N, _H), jnp.float32),
               jax.ShapeDtypeStruct((_N, _H), jnp.float32)],
)
_tc_post = pl.pallas_call(
    _tc_post_body,
    out_shape=[jax.ShapeDtypeStruct((_G, _C), jnp.float32),
               jax.ShapeDtypeStruct((_G, _H), jnp.float32)],
)


def kernel(x, edge_index, batch, W1, b1, bn1_g, bn1_b, W2, b2,
           cW1, cb1, cbn_g, cbn_b, cW2, cb2):
    src3 = edge_index[0].reshape(_NW, _NCHUNK, _CH)
    dst3 = edge_index[1].reshape(_NW, _NCHUNK, _CH)
    bat2 = batch.reshape(1, _N)
    ones_row = jnp.ones((_CH, _H), jnp.float32)
    zer_row = jnp.zeros((_RPT, _H), jnp.float32)

    degp = _sc_degree(dst3, ones_row, zer_row)
    u1, xs1 = _tc_pre(x, W1, bn1_g, degp)
    a_part = _sc_segsum(xs1, src3, dst3, zer_row)
    u2, xs2 = _tc_mid(a_part, u1, degp, W2, b1, bn1_g, bn1_b)
    b_part = _sc_segsum(xs2, src3, dst3, zer_row)
    pred, rep = _tc_post(b_part, u2, degp, b2, bat2,
                         cW1, cb1, cbn_g, cbn_b, cW2, cb2)
    return (pred, rep)


# speedup vs baseline: 2.3303x; 1.2019x over previous
import functools

import jax
import jax.numpy as jnp
from jax import lax
from jax.experimental import pallas as pl
from jax.experimental.pallas import tpu as pltpu
from jax.experimental.pallas import tpu_sc as plsc

_N = 10000
_E = 320000
_D = 128
_H = 128
_C = 16
_G = 64
_EPS = 1e-5

_NC = 2
_NS = 16
_NW = _NC * _NS
_EPW = _E // _NW
_CH = 125
_NCHUNK = 80
_GRP = 8
_NP = 10240
_RPT = _NP // _NS

_mesh = plsc.VectorSubcoreMesh(core_axis_name="c", subcore_axis_name="s")


@functools.partial(
    pl.kernel,
    out_type=jax.ShapeDtypeStruct((_NC, _NP, _H), jnp.float32),
    mesh=_mesh,
    scratch_types=[
        pltpu.VMEM((_NCHUNK, _CH), jnp.int32),
        pltpu.VMEM((_CH, _H), jnp.float32),
        pltpu.VMEM_SHARED((_NP, _H), jnp.float32),
    ],
)
def _sc_degree(dst_h, ones_h, zer_h, out_h, idx_v, ones_v, deg_sh):
    c = lax.axis_index("c")
    s = lax.axis_index("s")
    w = c * _NS + s
    pltpu.sync_copy(zer_h, deg_sh.at[pl.ds(s * _RPT, _RPT)])
    pltpu.sync_copy(ones_h, ones_v)
    pltpu.sync_copy(dst_h.at[w], idx_v)
    plsc.subcore_barrier()

    def step(j, carry):
        pltpu.sync_copy(ones_v, deg_sh.at[idx_v.at[j]], add=True)
        return carry

    lax.fori_loop(0, _NCHUNK, step, 0)
    plsc.subcore_barrier()
    pltpu.sync_copy(deg_sh.at[pl.ds(s * _RPT, _RPT)],
                    out_h.at[c, pl.ds(s * _RPT, _RPT)])


@functools.partial(
    pl.kernel,
    out_type=jax.ShapeDtypeStruct((_NC, _NP, _H), jnp.float32),
    mesh=_mesh,
    scratch_types=[
        pltpu.VMEM((_NCHUNK, _CH), jnp.int32),
        pltpu.VMEM((_GRP, _CH), jnp.int32),
        pltpu.VMEM((_CH, _H), jnp.float32),
        pltpu.VMEM((_CH, _H), jnp.float32),
        pltpu.VMEM_SHARED((_NP, _H), jnp.float32),
        pltpu.SemaphoreType.DMA,
        pltpu.SemaphoreType.DMA,
        pltpu.SemaphoreType.DMA,
    ],
)
def _sc_segsum(xs_h, src_h, dst_h, zer_h, out_h,
               sidx, dring, rows0, rows1, acc_sh, gsem, ssem0, ssem1):
    c = lax.axis_index("c")
    s = lax.axis_index("s")
    w = c * _NS + s
    pltpu.sync_copy(zer_h, acc_sh.at[pl.ds(s * _RPT, _RPT)])
    pltpu.sync_copy(src_h.at[w], sidx)
    plsc.subcore_barrier()
    pltpu.async_copy(xs_h.at[sidx.at[0]], rows0, gsem).wait()

    def step(k, carry):
        base = k * _GRP
        pltpu.sync_copy(dst_h.at[w, pl.ds(base, _GRP)], dring)
        for p in range(_GRP // 2):
            j1 = base + 2 * p + 1
            s0 = pltpu.async_copy(rows0, acc_sh.at[dring.at[2 * p]],
                                  ssem0, add=True)
            pltpu.async_copy(xs_h.at[sidx.at[j1]], rows1, gsem).wait()
            s1 = pltpu.async_copy(rows1, acc_sh.at[dring.at[2 * p + 1]],
                                  ssem1, add=True)
            s0.wait()
            if p < _GRP // 2 - 1:
                pltpu.async_copy(xs_h.at[sidx.at[j1 + 1]], rows0,
                                 gsem).wait()
            else:
                @pl.when(k < _NCHUNK // _GRP - 1)
                def _():
                    pltpu.async_copy(xs_h.at[sidx.at[j1 + 1]], rows0,
                                     gsem).wait()
            s1.wait()
        return carry

    lax.fori_loop(0, _NCHUNK // _GRP, step, 0)
    plsc.subcore_barrier()
    pltpu.sync_copy(acc_sh.at[pl.ds(s * _RPT, _RPT)],
                    out_h.at[c, pl.ds(s * _RPT, _RPT)])


def _dinv_from_partials(degp):
    deg = degp[0, :_N, 0:1] + degp[1, :_N, 0:1] + 1.0
    return lax.rsqrt(deg)


def _tc_pre_body(x_ref, w1_ref, g1_ref, degp_ref, u1_ref, xs1_ref):
    s1 = g1_ref[:] * (1.0 / jnp.sqrt(1.0 + _EPS))
    w1f = w1_ref[:] * s1[None, :]
    u1 = jnp.dot(x_ref[:], w1f, preferred_element_type=jnp.float32)
    dinv = _dinv_from_partials(degp_ref[:])
    u1_ref[:] = u1
    xs1_ref[:] = u1 * dinv


def _tc_mid_body(a_ref, u1_ref, degp_ref, w2_ref, b1_ref, g1_ref, bb1_ref,
                 u2_ref, xs2_ref):
    dinv = _dinv_from_partials(degp_ref[:])
    s1 = g1_ref[:] * (1.0 / jnp.sqrt(1.0 + _EPS))
    b1f = b1_ref[:] * s1 + bb1_ref[:]
    agg = dinv * (a_ref[0, :_N] + a_ref[1, :_N]) + (dinv * dinv) * u1_ref[:]
    h = jnp.maximum(agg + b1f[None, :], 0.0)
    u2 = jnp.dot(h, w2_ref[:], preferred_element_type=jnp.float32)
    u2_ref[:] = u2
    xs2_ref[:] = u2 * dinv


def _tc_post_body(b_ref, u2_ref, degp_ref, b2_ref, bat_ref,
                  cw1_ref, cb1_ref, cg_ref, cbb_ref, cw2_ref, cb2_ref,
                  pred_ref, rep_ref):
    dinv = _dinv_from_partials(degp_ref[:])
    emb = (dinv * (b_ref[0, :_N] + b_ref[1, :_N]) + (dinv * dinv) * u2_ref[:]
           + b2_ref[:][None, :])
    seg = lax.broadcasted_iota(jnp.int32, (_G, _N), 0)
    oh = (seg == bat_ref[:]).astype(jnp.float32)
    counts = jnp.sum(oh, axis=1, keepdims=True)
    sums = jnp.dot(oh, emb, preferred_element_type=jnp.float32)
    rep = sums / jnp.maximum(counts, 1.0)
    rep = rep / jnp.sqrt(jnp.sum(rep * rep, axis=1, keepdims=True))
    cs = cg_ref[:] * (1.0 / jnp.sqrt(1.0 + _EPS))
    z = jnp.dot(rep, cw1_ref[:], preferred_element_type=jnp.float32)
    z = jnp.maximum((z + cb1_ref[:][None, :]) * cs[None, :] + cbb_ref[:][None, :], 0.0)
    z = jnp.dot(z, cw2_ref[:], preferred_element_type=jnp.float32) + cb2_ref[:][None, :]
    m = jnp.max(z, axis=1, keepdims=True)
    lse = m + jnp.log(jnp.sum(jnp.exp(z - m), axis=1, keepdims=True))
    pred_ref[:] = z - lse
    rep_ref[:] = rep


_tc_pre = pl.pallas_call(
    _tc_pre_body,
    out_shape=[jax.ShapeDtypeStruct((_N, _H), jnp.float32),
               jax.ShapeDtypeStruct((_N, _H), jnp.float32)],
)
_tc_mid = pl.pallas_call(
    _tc_mid_body,
    out_shape=[jax.ShapeDtypeStruct((_N, _H), jnp.float32),
               jax.ShapeDtypeStruct((_N, _H), jnp.float32)],
)
_tc_post = pl.pallas_call(
    _tc_post_body,
    out_shape=[jax.ShapeDtypeStruct((_G, _C), jnp.float32),
               jax.ShapeDtypeStruct((_G, _H), jnp.float32)],
)


def kernel(x, edge_index, batch, W1, b1, bn1_g, bn1_b, W2, b2,
           cW1, cb1, cbn_g, cbn_b, cW2, cb2):
    src3 = edge_index[0].reshape(_NW, _NCHUNK, _CH)
    dst3 = edge_index[1].reshape(_NW, _NCHUNK, _CH)
    bat2 = batch.reshape(1, _N)
    ones_row = jnp.ones((_CH, _H), jnp.float32)
    zer_row = jnp.zeros((_RPT, _H), jnp.float32)

    degp = _sc_degree(dst3, ones_row, zer_row)
    u1, xs1 = _tc_pre(x, W1, bn1_g, degp)
    a_part = _sc_segsum(xs1, src3, dst3, zer_row)
    u2, xs2 = _tc_mid(a_part, u1, degp, W2, b1, bn1_g, bn1_b)
    b_part = _sc_segsum(xs2, src3, dst3, zer_row)
    pred, rep = _tc_post(b_part, u2, degp, b2, bat2,
                         cW1, cb1, cbn_g, cbn_b, cW2, cb2)
    return (pred, rep)
